# baseline (speedup 1.0000x reference)
import jax
import jax.numpy as jnp
from jax import lax
from jax.experimental import pallas as pl
from jax.experimental.pallas import tpu as pltpu
from jax.experimental.pallas import tpu_sc as plsc

B = 16384
LANES = 16
NUM_TILES = 16
ROWS_PER_TILE = B // NUM_TILES
STEPS = ROWS_PER_TILE // LANES


def _f1_body(out_hbm, tgt_hbm, res_hbm, logits_v, tgt_v, part_v, shared,
             acc_v, out_v, sem_a, sem_b):
    tid = lax.axis_index("s")

    cp_a = pltpu.async_copy(
        out_hbm.at[pl.ds(tid * 2 * ROWS_PER_TILE, 2 * ROWS_PER_TILE)],
        logits_v, sem_a)
    cp_b = pltpu.async_copy(
        tgt_hbm.at[pl.ds(tid * ROWS_PER_TILE, ROWS_PER_TILE)], tgt_v, sem_b)
    cp_a.wait()
    cp_b.wait()

    lane = lax.iota(jnp.int32, LANES)
    hi = lane >= 8
    g1_base = lane * 2 + hi.astype(jnp.int32)
    g2_base = lane * 2 + (1 - hi.astype(jnp.int32))
    zero = jnp.zeros((LANES,), jnp.float32)

    def step(i, carry):
        tp, pred_s, pos_s = carry
        off = i * (2 * LANES)
        g1 = plsc.load_gather(logits_v, [g1_base + off])
        g2 = plsc.load_gather(logits_v, [g2_base + off])
        t = tgt_v[pl.ds(i * LANES, LANES)]
        pred = jnp.where(hi, g1 > g2, g2 > g1)
        pf = pred.astype(jnp.float32)
        tf = t.astype(jnp.float32)
        return tp + pf * tf, pred_s + pf, pos_s + tf

    tp, pred_s, pos_s = lax.fori_loop(0, STEPS, step, (zero, zero, zero),
                                      unroll=16)

    tp_s = lax.broadcast_in_dim(jnp.sum(tp), (LANES,), ())
    part_v[pl.ds(0, LANES)] = tp_s
    part_v[pl.ds(LANES, LANES)] = (
        lax.broadcast_in_dim(jnp.sum(pred_s), (LANES,), ()) - tp_s)
    part_v[pl.ds(2 * LANES, LANES)] = (
        lax.broadcast_in_dim(jnp.sum(pos_s), (LANES,), ()) - tp_s)
    pltpu.sync_copy(part_v, shared.at[pl.ds(tid * 3 * LANES, 3 * LANES)])
    plsc.subcore_barrier()

    @pl.when(tid == 0)
    def _():
        pltpu.sync_copy(shared, acc_v)
        TP = jnp.zeros((LANES,), jnp.float32)
        FP = jnp.zeros((LANES,), jnp.float32)
        FN = jnp.zeros((LANES,), jnp.float32)
        for t in range(NUM_TILES):
            TP = TP + acc_v[pl.ds((3 * t) * LANES, LANES)]
            FP = FP + acc_v[pl.ds((3 * t + 1) * LANES, LANES)]
            FN = FN + acc_v[pl.ds((3 * t + 2) * LANES, LANES)]
        precision = TP / (TP + FP + 1e-10)
        recall = TP / (TP + FN + 1e-10)
        f1 = 2.0 * precision * recall / (precision + recall + 1e-10)
        out_v[...] = f1
        pltpu.sync_copy(out_v, res_hbm)


@jax.jit
def _f1_sc(out_flat, tgt):
    mesh = plsc.VectorSubcoreMesh(core_axis_name="c", subcore_axis_name="s",
                                  num_cores=1, num_subcores=NUM_TILES)
    run = pl.kernel(
        _f1_body,
        out_type=jax.ShapeDtypeStruct((LANES,), jnp.float32),
        mesh=mesh,
        scratch_types=[
            pltpu.VMEM((2 * ROWS_PER_TILE,), jnp.float32),
            pltpu.VMEM((ROWS_PER_TILE,), jnp.int32),
            pltpu.VMEM((3 * LANES,), jnp.float32),
            pltpu.VMEM_SHARED((NUM_TILES * 3 * LANES,), jnp.float32),
            pltpu.VMEM((NUM_TILES * 3 * LANES,), jnp.float32),
            pltpu.VMEM((LANES,), jnp.float32),
            pltpu.SemaphoreType.DMA,
            pltpu.SemaphoreType.DMA,
        ],
        compiler_params=pltpu.CompilerParams(needs_layout_passes=False),
    )
    return run(out_flat, tgt)


def kernel(output, target):
    out_flat = output.reshape(-1)
    tgt = target.astype(jnp.int32)
    res = _f1_sc(out_flat, tgt)
    return res[0]


# speedup vs baseline: 1.0105x; 1.0105x over previous
import jax
import jax.numpy as jnp
from jax import lax
from jax.experimental import pallas as pl
from jax.experimental.pallas import tpu as pltpu
from jax.experimental.pallas import tpu_sc as plsc

B = 16384
LANES = 16
NUM_TILES = 16
ROWS_PER_TILE = B // NUM_TILES
STEPS = ROWS_PER_TILE // LANES


def _f1_body(out_hbm, tgt_hbm, res_hbm, logits_v, tgt_v, part_v, shared,
             acc_v, out_v, sem_a, sem_b):
    tid = lax.axis_index("s")

    cp_a = pltpu.async_copy(
        out_hbm.at[pl.ds(tid * 2 * ROWS_PER_TILE, 2 * ROWS_PER_TILE)],
        logits_v, sem_a)
    cp_b = pltpu.async_copy(
        tgt_hbm.at[pl.ds(tid * ROWS_PER_TILE, ROWS_PER_TILE)], tgt_v, sem_b)
    cp_a.wait()
    cp_b.wait()

    even = lax.iota(jnp.int32, LANES) * 2
    zero = jnp.zeros((LANES,), jnp.float32)

    def step(i, carry):
        tp, pp = carry
        idx = even + i * (2 * LANES)
        a = plsc.load_gather(logits_v, [idx])
        b = plsc.load_gather(logits_v, [idx + 1])
        t = tgt_v[pl.ds(i * LANES, LANES)]
        pf = (b > a).astype(jnp.float32)
        tf = t.astype(jnp.float32)
        return tp + pf * tf, pp + (pf + tf)

    tp, pp = lax.fori_loop(0, STEPS, step, (zero, zero), unroll=4)

    part_v[pl.ds(0, LANES)] = lax.broadcast_in_dim(jnp.sum(tp), (LANES,), ())
    part_v[pl.ds(LANES, LANES)] = lax.broadcast_in_dim(jnp.sum(pp),
                                                       (LANES,), ())
    pltpu.sync_copy(part_v, shared.at[pl.ds(tid * 2 * LANES, 2 * LANES)])
    plsc.subcore_barrier()

    @pl.when(tid == 0)
    def _():
        pltpu.sync_copy(shared, acc_v)
        TP = jnp.zeros((LANES,), jnp.float32)
        PP = jnp.zeros((LANES,), jnp.float32)
        for t in range(NUM_TILES):
            TP = TP + acc_v[pl.ds((2 * t) * LANES, LANES)]
            PP = PP + acc_v[pl.ds((2 * t + 1) * LANES, LANES)]
        out_v[...] = (2.0 * TP) / (PP + 1e-10)
        pltpu.sync_copy(out_v, res_hbm)


@jax.jit
def _f1_sc(out_flat, tgt):
    mesh = plsc.VectorSubcoreMesh(core_axis_name="c", subcore_axis_name="s",
                                  num_cores=1, num_subcores=NUM_TILES)
    run = pl.kernel(
        _f1_body,
        out_type=jax.ShapeDtypeStruct((LANES,), jnp.float32),
        mesh=mesh,
        scratch_types=[
            pltpu.VMEM((2 * ROWS_PER_TILE,), jnp.float32),
            pltpu.VMEM((ROWS_PER_TILE,), jnp.int32),
            pltpu.VMEM((2 * LANES,), jnp.float32),
            pltpu.VMEM_SHARED((NUM_TILES * 2 * LANES,), jnp.float32),
            pltpu.VMEM((NUM_TILES * 2 * LANES,), jnp.float32),
            pltpu.VMEM((LANES,), jnp.float32),
            pltpu.SemaphoreType.DMA,
            pltpu.SemaphoreType.DMA,
        ],
        compiler_params=pltpu.CompilerParams(needs_layout_passes=False),
    )
    return run(out_flat, tgt)


def kernel(output, target):
    out_flat = output.reshape(-1)
    tgt = target.astype(jnp.int32)
    res = _f1_sc(out_flat, tgt)
    return res[0]
